# baseline (speedup 1.0000x reference)
import functools
import jax
import jax.numpy as jnp
from jax import lax
from jax.experimental import pallas as pl
from jax.experimental.pallas import tpu as pltpu
from jax.experimental.pallas import tpu_sc as plsc

N = 10000
E = 320000
D = 128
H = 8
C = 16
DE = 16
DFF = 512

NPAD = 10240
HH = H // 2
ROWG = 80
ROWS = 80
K = 80
TILE_E = E // 16
NCHUNK = TILE_E // K
NB = 1000
EB = 4000


def _mm_body(x_ref, w_ref, o_ref):
    o_ref[...] = jnp.dot(x_ref[...], w_ref[...],
                         preferred_element_type=jnp.float32)


def _proj_body(x_ref, w_ref, oga_ref, oad_ref):
    t = jnp.dot(x_ref[...], w_ref[0], preferred_element_type=jnp.float32)
    oga_ref[0] = t[:, :ROWG]
    oad_ref[0] = t[:, ROWG:]


def _proj(nf, wcat):
    return pl.pallas_call(
        _proj_body,
        grid=(2, N // NB),
        in_specs=[pl.BlockSpec((NB, D), lambda j, i: (i, 0)),
                  pl.BlockSpec((1, D, ROWG + 16), lambda j, i: (j, 0, 0))],
        out_specs=[pl.BlockSpec((1, NB, ROWG), lambda j, i: (j, i, 0)),
                   pl.BlockSpec((1, NB, 16), lambda j, i: (j, i, 0))],
        out_shape=[jax.ShapeDtypeStruct((2, N, ROWG), jnp.float32),
                   jax.ShapeDtypeStruct((2, N, 16), jnp.float32)],
    )(nf, wcat)


def _ale_body(ew_ref, v1_ref, v2_ref, o1_ref, o2_ref):
    ewb = ew_ref[...]
    o1_ref[0] = jnp.dot(ewb, v1_ref[0], preferred_element_type=jnp.float32)
    o2_ref[0] = jnp.dot(ewb, v2_ref[0], preferred_element_type=jnp.float32)


def _ale_both(ew, ve1, ve2):
    return pl.pallas_call(
        _ale_body,
        grid=(2, E // EB),
        in_specs=[pl.BlockSpec((EB, DE), lambda j, i: (i, 0)),
                  pl.BlockSpec((1, DE, 16), lambda j, i: (j, 0, 0)),
                  pl.BlockSpec((1, DE, 16), lambda j, i: (j, 0, 0))],
        out_specs=[pl.BlockSpec((1, EB, 16), lambda j, i: (j, i, 0)),
                   pl.BlockSpec((1, EB, 16), lambda j, i: (j, i, 0))],
        out_shape=[jax.ShapeDtypeStruct((2, E, 16), jnp.float32),
                   jax.ShapeDtypeStruct((2, E, 16), jnp.float32)],
    )(ew, ve1, ve2)


def _ln_leaky(x, gg, bn):
    m = jnp.mean(x, axis=1, keepdims=True)
    xc = x - m
    s = jnp.sqrt(jnp.sum(xc * xc, axis=1, keepdims=True) / (D - 1))
    y = gg * xc / (s + 1e-6) + bn
    return jnp.where(y > 0, y, 0.01 * y)


def _post_core(p0_ref, p1_ref, g0_ref, g1_ref, a0_ref, a1_ref, nf_ref,
               b_ref, gg_ref, bn_ref, p8_ref):
    p0 = p0_ref[0]
    p1 = p1_ref[0]
    g0 = g0_ref[0]
    g1 = g1_ref[0]
    h = jnp.concatenate([g0[:, :64], g1[:, :64]], axis=1)
    als = jnp.concatenate([g0[:, 64:64 + HH], g1[:, 64:64 + HH]], axis=1)
    ald = jnp.concatenate([a0_ref[0][:, :HH], a1_ref[0][:, :HH]], axis=1)
    acc = jnp.concatenate([p0[:, :64], p1[:, :64]], axis=1)
    den_p = jnp.concatenate([p0[:, 64:64 + HH], p1[:, 64:64 + HH]], axis=1)
    acc_la = jnp.concatenate([p0[:, 68:68 + HH], p1[:, 68:68 + HH]], axis=1)
    deg = p0[:, 72:73]
    ale_loop = acc_la / jnp.maximum(deg, 1.0)
    al = als + ald + ale_loop
    al = jnp.where(al > 0, al, 0.2 * al)
    exl = jnp.exp(al)
    rden = 1.0 / (den_p + exl + 1e-16)
    p8 = p8_ref[...]
    exl128 = jnp.dot(exl, p8, preferred_element_type=jnp.float32)
    rden128 = jnp.dot(rden, p8, preferred_element_type=jnp.float32)
    a1 = (acc + exl128 * h) * rden128 + b_ref[...]
    return nf_ref[...] + _ln_leaky(a1, gg_ref[...], bn_ref[...])


def _post_body(p0_ref, p1_ref, g0_ref, g1_ref, a0_ref, a1_ref, nf_ref,
               b_ref, gg_ref, bn_ref, p8_ref, o_ref):
    o_ref[...] = _post_core(p0_ref, p1_ref, g0_ref, g1_ref, a0_ref, a1_ref,
                            nf_ref, b_ref, gg_ref, bn_ref, p8_ref)


def _post_ffn_body(p0_ref, p1_ref, g0_ref, g1_ref, a0_ref, a1_ref, nf_ref,
                   b_ref, gg_ref, bn_ref, p8_ref,
                   w1_ref, b1_ref, w2_ref, b2_ref, gg3_ref, bn3_ref, o_ref):
    nf2 = _post_core(p0_ref, p1_ref, g0_ref, g1_ref, a0_ref, a1_ref,
                     nf_ref, b_ref, gg_ref, bn_ref, p8_ref)
    t = jnp.dot(nf2, w1_ref[...], preferred_element_type=jnp.float32)
    t = jnp.maximum(t + b1_ref[...], 0.0)
    ff = jnp.dot(t, w2_ref[...], preferred_element_type=jnp.float32)
    ff = ff + b2_ref[...]
    o_ref[...] = nf2 + _ln_leaky(ff, gg3_ref[...], bn3_ref[...])


def _post_specs():
    return [pl.BlockSpec((1, NB, ROWS), lambda i: (0, i, 0)),
            pl.BlockSpec((1, NB, ROWS), lambda i: (1, i, 0)),
            pl.BlockSpec((1, NB, ROWG), lambda i: (0, i, 0)),
            pl.BlockSpec((1, NB, ROWG), lambda i: (1, i, 0)),
            pl.BlockSpec((1, NB, 16), lambda i: (0, i, 0)),
            pl.BlockSpec((1, NB, 16), lambda i: (1, i, 0)),
            pl.BlockSpec((NB, D), lambda i: (i, 0)),
            pl.BlockSpec((1, D), lambda i: (0, 0)),
            pl.BlockSpec((1, D), lambda i: (0, 0)),
            pl.BlockSpec((1, D), lambda i: (0, 0)),
            pl.BlockSpec((H, D), lambda i: (0, 0))]


def _post(part, ga, ad, nf, b, gg, bn, p8):
    return pl.pallas_call(
        _post_body,
        grid=(N // NB,),
        in_specs=_post_specs(),
        out_specs=pl.BlockSpec((NB, D), lambda i: (i, 0)),
        out_shape=jax.ShapeDtypeStruct((N, D), jnp.float32),
    )(part, part, ga, ga, ad, ad, nf, b, gg, bn, p8)


def _post_ffn(part, ga, ad, nf, b, gg, bn, p8, w1, b1, w2, b2, gg3, bn3):
    specs = _post_specs() + [
        pl.BlockSpec((D, DFF), lambda i: (0, 0)),
        pl.BlockSpec((1, DFF), lambda i: (0, 0)),
        pl.BlockSpec((DFF, D), lambda i: (0, 0)),
        pl.BlockSpec((1, D), lambda i: (0, 0)),
        pl.BlockSpec((1, D), lambda i: (0, 0)),
        pl.BlockSpec((1, D), lambda i: (0, 0))]
    return pl.pallas_call(
        _post_ffn_body,
        grid=(N // NB,),
        in_specs=specs,
        out_specs=pl.BlockSpec((NB, D), lambda i: (i, 0)),
        out_shape=jax.ShapeDtypeStruct((N, D), jnp.float32),
    )(part, part, ga, ga, ad, ad, nf, b, gg, bn, p8, w1, b1, w2, b2, gg3, bn3)


def _sc_body(src3_h, dst3_h, ale_h, ga_h, ad_h, part_h,
             srct_v, dstt_v, grows_v, adst_v, alev_v, s_v, acc_sh,
             sem_in0, sem_in1, sem_sc0, sem_sc1):
    cid = lax.axis_index("c")
    sid = lax.axis_index("s")
    lanes = lax.iota(jnp.int32, 16)
    zv = jnp.zeros((16,), jnp.float32)
    one8v = jnp.where(lanes == 8, 1.0, 0.0).astype(jnp.float32)
    lo4 = lanes < 4
    lo8 = lanes < 8
    sem_in = (sem_in0, sem_in1)
    sem_sc = (sem_sc0, sem_sc1)
    estart = sid * TILE_E
    my_ga = ga_h.at[cid]
    my_ad = ad_h.at[cid]
    my_ale = ale_h.at[cid]

    pltpu.sync_copy(src3_h.at[sid], srct_v)
    pltpu.sync_copy(dst3_h.at[sid], dstt_v)

    z_v = s_v.at[0]

    def zrow(i, _):
        r = i // (ROWS // 16)
        col = (i % (ROWS // 16)) * 16
        z_v[r, pl.ds(col, 16)] = zv
        return 0
    lax.fori_loop(0, K * (ROWS // 16), zrow, 0)

    def zcopy(j, _):
        pltpu.sync_copy(z_v, acc_sh.at[pl.ds(sid * 640 + j * K, K)])
        return 0
    lax.fori_loop(0, 640 // K, zcopy, 0)
    plsc.subcore_barrier()

    def issue_in(c, p):
        base = estart + c * K
        pltpu.async_copy(my_ale.at[pl.ds(base, K)], alev_v.at[p], sem_in[p])
        pltpu.async_copy(my_ga.at[srct_v.at[c]], grows_v.at[p], sem_in[p])
        pltpu.async_copy(my_ad.at[dstt_v.at[c]], adst_v.at[p], sem_in[p])

    def wait_in(c, p):
        base = estart + c * K
        pltpu.make_async_copy(my_ale.at[pl.ds(base, K)], alev_v.at[p],
                              sem_in[p]).wait()
        pltpu.make_async_copy(my_ga.at[srct_v.at[c]], grows_v.at[p],
                              sem_in[p]).wait()
        pltpu.make_async_copy(my_ad.at[dstt_v.at[c]], adst_v.at[p],
                              sem_in[p]).wait()

    def do_scatter(c, p):
        pltpu.async_copy(s_v.at[p], acc_sh.at[dstt_v.at[c]], sem_sc[p],
                         add=True)

    def wait_sc(c, p):
        pltpu.make_async_copy(s_v.at[p], acc_sh.at[dstt_v.at[c]],
                              sem_sc[p]).wait()

    dnums = lax.GatherDimensionNumbers(
        offset_dims=(), collapsed_slice_dims=(0,), start_index_map=(0,))
    bcast_idx = [jnp.full((16, 1), hh, jnp.int32) for hh in range(HH)]

    def compute(p):
        gp = grows_v.at[p]
        ap = adst_v.at[p]
        lp = alev_v.at[p]
        sp = s_v.at[p]

        def edge(e, _):
            alev = lp[e, :]
            av = gp[e, pl.ds(64, 16)] + ap[e, :] + alev
            al = jnp.where(av > 0, av, 0.2 * av)
            ex = jnp.exp(al)
            mix = jnp.where(lo4, ex, jnp.where(lo8, alev, one8v))
            sp[e, pl.ds(64, 16)] = mix
            for hh in range(HH):
                exb = lax.gather(ex, bcast_idx[hh], dnums, slice_sizes=(1,),
                                 mode=lax.GatherScatterMode.PROMISE_IN_BOUNDS)
                sp[e, pl.ds(hh * 16, 16)] = gp[e, pl.ds(hh * 16, 16)] * exb
            return 0
        lax.fori_loop(0, K, edge, 0, unroll=4)

    issue_in(0, 0)
    issue_in(1, 1)
    for c, p in ((0, 0), (1, 1)):
        wait_in(c, p)
        compute(p)
        do_scatter(c, p)
        issue_in(c + 2, p)

    def pair(i, _):
        for off, p in ((0, 0), (1, 1)):
            c = 2 * i + off
            wait_in(c, p)
            wait_sc(c - 2, p)
            compute(p)
            do_scatter(c, p)
            issue_in(c + 2, p)
        return 0
    lax.fori_loop(1, NCHUNK // 2 - 1, pair, 0)

    for c, p in ((NCHUNK - 2, 0), (NCHUNK - 1, 1)):
        wait_in(c, p)
        wait_sc(c - 2, p)
        compute(p)
        do_scatter(c, p)
    wait_sc(NCHUNK - 2, 0)
    wait_sc(NCHUNK - 1, 1)

    plsc.subcore_barrier()
    pltpu.sync_copy(acc_sh.at[pl.ds(sid * 640, 640)],
                    part_h.at[cid, pl.ds(sid * 640, 640)])


def _sc_edge_pass(src3, dst3, ale, ga, ad):
    mesh = plsc.VectorSubcoreMesh(core_axis_name="c", subcore_axis_name="s")
    f = pl.kernel(
        _sc_body,
        mesh=mesh,
        compiler_params=pltpu.CompilerParams(use_tc_tiling_on_sc=False),
        out_type=jax.ShapeDtypeStruct((2, NPAD, ROWS), jnp.float32),
        scratch_types=[
            pltpu.VMEM((NCHUNK, K), jnp.int32),
            pltpu.VMEM((NCHUNK, K), jnp.int32),
            pltpu.VMEM((2, K, ROWG), jnp.float32),
            pltpu.VMEM((2, K, 16), jnp.float32),
            pltpu.VMEM((2, K, 16), jnp.float32),
            pltpu.VMEM((2, K, ROWS), jnp.float32),
            pltpu.VMEM_SHARED((NPAD, ROWS), jnp.float32),
            pltpu.SemaphoreType.DMA,
            pltpu.SemaphoreType.DMA,
            pltpu.SemaphoreType.DMA,
            pltpu.SemaphoreType.DMA,
        ],
    )
    return f(src3, dst3, ale, ga, ad)


def _dup4(a):
    return jnp.concatenate([a, a, a, a], axis=1)


def _prep_w(W, a_src, a_dst):
    w3 = W.reshape(D, H, C)
    ws = jnp.einsum('dhc,hc->dh', w3, a_src)
    wd = jnp.einsum('dhc,hc->dh', w3, a_dst)
    planes = []
    for c in range(2):
        planes.append(jnp.concatenate(
            [W[:, 64 * c:64 * c + 64],
             _dup4(ws[:, HH * c:HH * c + HH]),
             _dup4(wd[:, HH * c:HH * c + HH])], axis=1))
    return jnp.stack(planes)


def _prep_ve(We, a_e):
    ve = jnp.einsum('dhc,hc->dh', We.reshape(DE, H, C), a_e)
    return jnp.stack([_dup4(ve[:, :HH]), _dup4(ve[:, HH:])])


def kernel(nf, ei, ew, W1, as1, ad1, We1, ae1, b1, W2, as2, ad2, We2, ae2, b2,
           g1, bn1, g2, bn2, g3, bn3, Wf1, bf1, Wf2, bf2):
    src3 = ei[0].reshape(16, NCHUNK, K)
    dst3 = ei[1].reshape(16, NCHUNK, K)
    wcat1 = _prep_w(W1, as1, ad1)
    wcat2 = _prep_w(W2, as2, ad2)
    ale1, ale2 = _ale_both(ew, _prep_ve(We1, ae1), _prep_ve(We2, ae2))
    p8 = jnp.repeat(jnp.eye(H, dtype=jnp.float32), C, axis=1)

    ga1, ad1_t = _proj(nf, wcat1)
    part1 = _sc_edge_pass(src3, dst3, ale1, ga1, ad1_t)
    nf = _post(part1, ga1, ad1_t, nf,
               b1.reshape(1, D), g1.reshape(1, D), bn1.reshape(1, D), p8)

    ga2, ad2_t = _proj(nf, wcat2)
    part2 = _sc_edge_pass(src3, dst3, ale2, ga2, ad2_t)
    nf = _post_ffn(part2, ga2, ad2_t, nf,
                   b2.reshape(1, D), g2.reshape(1, D), bn2.reshape(1, D), p8,
                   Wf1, bf1.reshape(1, DFF), Wf2, bf2.reshape(1, D),
                   g3.reshape(1, D), bn3.reshape(1, D))
    return nf


# speedup vs baseline: 1.7074x; 1.7074x over previous
import functools
import jax
import jax.numpy as jnp
from jax import lax
from jax.experimental import pallas as pl
from jax.experimental.pallas import tpu as pltpu
from jax.experimental.pallas import tpu_sc as plsc

N = 10000
E = 320000
D = 128
H = 8
C = 16
DE = 16
DFF = 512

NPAD = 10240
HH = H // 2
ROWG = 96
ROWS = 80
K = 80
TILE_E = E // 16
NCHUNK = TILE_E // K
NB = 1000
EB = 4000


def _mm_body(x_ref, w_ref, o_ref):
    o_ref[...] = jnp.dot(x_ref[...], w_ref[...],
                         preferred_element_type=jnp.float32)


def _mm3_body(x_ref, w_ref, o_ref):
    o_ref[0] = jnp.dot(x_ref[...], w_ref[0],
                       preferred_element_type=jnp.float32)


def _proj(nf, wcat):
    return pl.pallas_call(
        _mm3_body,
        grid=(2, N // NB),
        in_specs=[pl.BlockSpec((NB, D), lambda j, i: (i, 0)),
                  pl.BlockSpec((1, D, ROWG), lambda j, i: (j, 0, 0))],
        out_specs=pl.BlockSpec((1, NB, ROWG), lambda j, i: (j, i, 0)),
        out_shape=jax.ShapeDtypeStruct((2, N, ROWG), jnp.float32),
    )(nf, wcat)


def _ale_body(ew_ref, v1_ref, v2_ref, o1_ref, o2_ref):
    ewb = ew_ref[...]
    o1_ref[0] = jnp.dot(ewb, v1_ref[0], preferred_element_type=jnp.float32)
    o2_ref[0] = jnp.dot(ewb, v2_ref[0], preferred_element_type=jnp.float32)


def _ale_both(ew, ve1, ve2):
    return pl.pallas_call(
        _ale_body,
        grid=(2, E // EB),
        in_specs=[pl.BlockSpec((EB, DE), lambda j, i: (i, 0)),
                  pl.BlockSpec((1, DE, 16), lambda j, i: (j, 0, 0)),
                  pl.BlockSpec((1, DE, 16), lambda j, i: (j, 0, 0))],
        out_specs=[pl.BlockSpec((1, EB, 16), lambda j, i: (j, i, 0)),
                   pl.BlockSpec((1, EB, 16), lambda j, i: (j, i, 0))],
        out_shape=[jax.ShapeDtypeStruct((2, E, 16), jnp.float32),
                   jax.ShapeDtypeStruct((2, E, 16), jnp.float32)],
    )(ew, ve1, ve2)


def _ln_leaky(x, gg, bn):
    m = jnp.mean(x, axis=1, keepdims=True)
    xc = x - m
    s = jnp.sqrt(jnp.sum(xc * xc, axis=1, keepdims=True) / (D - 1))
    y = gg * xc / (s + 1e-6) + bn
    return jnp.where(y > 0, y, 0.01 * y)


def _post_core(p0_ref, p1_ref, g0_ref, g1_ref, nf_ref,
               b_ref, gg_ref, bn_ref, p8_ref):
    p0 = p0_ref[0]
    p1 = p1_ref[0]
    g0 = g0_ref[0]
    g1 = g1_ref[0]
    h = jnp.concatenate([g0[:, :64], g1[:, :64]], axis=1)
    als = jnp.concatenate([g0[:, 64:64 + HH], g1[:, 64:64 + HH]], axis=1)
    ald = jnp.concatenate([g0[:, 80:80 + HH], g1[:, 80:80 + HH]], axis=1)
    acc = jnp.concatenate([p0[:, :64], p1[:, :64]], axis=1)
    den_p = jnp.concatenate([p0[:, 64:64 + HH], p1[:, 64:64 + HH]], axis=1)
    acc_la = jnp.concatenate([p0[:, 68:68 + HH], p1[:, 68:68 + HH]], axis=1)
    deg = p0[:, 72:73]
    ale_loop = acc_la / jnp.maximum(deg, 1.0)
    al = als + ald + ale_loop
    al = jnp.where(al > 0, al, 0.2 * al)
    exl = jnp.exp(al)
    rden = 1.0 / (den_p + exl + 1e-16)
    p8 = p8_ref[...]
    exl128 = jnp.dot(exl, p8, preferred_element_type=jnp.float32)
    rden128 = jnp.dot(rden, p8, preferred_element_type=jnp.float32)
    a1 = (acc + exl128 * h) * rden128 + b_ref[...]
    return nf_ref[...] + _ln_leaky(a1, gg_ref[...], bn_ref[...])


def _post_body(p0_ref, p1_ref, g0_ref, g1_ref, nf_ref,
               b_ref, gg_ref, bn_ref, p8_ref, o_ref):
    o_ref[...] = _post_core(p0_ref, p1_ref, g0_ref, g1_ref,
                            nf_ref, b_ref, gg_ref, bn_ref, p8_ref)


def _post_ffn_body(p0_ref, p1_ref, g0_ref, g1_ref, nf_ref,
                   b_ref, gg_ref, bn_ref, p8_ref,
                   w1_ref, b1_ref, w2_ref, b2_ref, gg3_ref, bn3_ref, o_ref):
    nf2 = _post_core(p0_ref, p1_ref, g0_ref, g1_ref,
                     nf_ref, b_ref, gg_ref, bn_ref, p8_ref)
    t = jnp.dot(nf2, w1_ref[...], preferred_element_type=jnp.float32)
    t = jnp.maximum(t + b1_ref[...], 0.0)
    ff = jnp.dot(t, w2_ref[...], preferred_element_type=jnp.float32)
    ff = ff + b2_ref[...]
    o_ref[...] = nf2 + _ln_leaky(ff, gg3_ref[...], bn3_ref[...])


def _post_specs():
    return [pl.BlockSpec((1, NB, ROWS), lambda i: (0, i, 0)),
            pl.BlockSpec((1, NB, ROWS), lambda i: (1, i, 0)),
            pl.BlockSpec((1, NB, ROWG), lambda i: (0, i, 0)),
            pl.BlockSpec((1, NB, ROWG), lambda i: (1, i, 0)),
            pl.BlockSpec((NB, D), lambda i: (i, 0)),
            pl.BlockSpec((1, D), lambda i: (0, 0)),
            pl.BlockSpec((1, D), lambda i: (0, 0)),
            pl.BlockSpec((1, D), lambda i: (0, 0)),
            pl.BlockSpec((H, D), lambda i: (0, 0))]


def _post(part, ga, nf, b, gg, bn, p8):
    return pl.pallas_call(
        _post_body,
        grid=(N // NB,),
        in_specs=_post_specs(),
        out_specs=pl.BlockSpec((NB, D), lambda i: (i, 0)),
        out_shape=jax.ShapeDtypeStruct((N, D), jnp.float32),
    )(part, part, ga, ga, nf, b, gg, bn, p8)


def _post_ffn(part, ga, nf, b, gg, bn, p8, w1, b1, w2, b2, gg3, bn3):
    specs = _post_specs() + [
        pl.BlockSpec((D, DFF), lambda i: (0, 0)),
        pl.BlockSpec((1, DFF), lambda i: (0, 0)),
        pl.BlockSpec((DFF, D), lambda i: (0, 0)),
        pl.BlockSpec((1, D), lambda i: (0, 0)),
        pl.BlockSpec((1, D), lambda i: (0, 0)),
        pl.BlockSpec((1, D), lambda i: (0, 0))]
    return pl.pallas_call(
        _post_ffn_body,
        grid=(N // NB,),
        in_specs=specs,
        out_specs=pl.BlockSpec((NB, D), lambda i: (i, 0)),
        out_shape=jax.ShapeDtypeStruct((N, D), jnp.float32),
    )(part, part, ga, ga, nf, b, gg, bn, p8, w1, b1, w2, b2, gg3, bn3)


def _sc_body(src3_h, dst3_h, ale_h, ga_h, ad_h, part_h,
             srct_v, dstt_v, grows_v, adst_v, alev_v, s_v, acc_sh,
             sem_in0, sem_in1, sem_sc0, sem_sc1):
    cid = lax.axis_index("c")
    sid = lax.axis_index("s")
    lanes = lax.iota(jnp.int32, 16)
    zv = jnp.zeros((16,), jnp.float32)
    one8v = jnp.where(lanes == 8, 1.0, 0.0).astype(jnp.float32)
    lo4 = lanes < 4
    lo8 = lanes < 8
    sem_in = (sem_in0, sem_in1)
    sem_sc = (sem_sc0, sem_sc1)
    estart = sid * TILE_E
    my_ga = ga_h.at[cid]
    my_ad = ad_h.at[cid]
    my_ale = ale_h.at[cid]

    pltpu.sync_copy(src3_h.at[sid], srct_v)
    pltpu.sync_copy(dst3_h.at[sid], dstt_v)

    z_v = s_v.at[0]

    def zrow(i, _):
        r = i // (ROWS // 16)
        col = (i % (ROWS // 16)) * 16
        z_v[r, pl.ds(col, 16)] = zv
        return 0
    lax.fori_loop(0, K * (ROWS // 16), zrow, 0)

    def zcopy(j, _):
        pltpu.sync_copy(z_v, acc_sh.at[pl.ds(sid * 640 + j * K, K)])
        return 0
    lax.fori_loop(0, 640 // K, zcopy, 0)
    plsc.subcore_barrier()

    def issue_in(c, p):
        base = estart + c * K
        pltpu.async_copy(my_ale.at[pl.ds(base, K)], alev_v.at[p], sem_in[p])
        pltpu.async_copy(my_ga.at[srct_v.at[c]], grows_v.at[p], sem_in[p])
        pltpu.async_copy(my_ad.at[dstt_v.at[c]], adst_v.at[p], sem_in[p])

    def wait_in(c, p):
        base = estart + c * K
        pltpu.make_async_copy(my_ale.at[pl.ds(base, K)], alev_v.at[p],
                              sem_in[p]).wait()
        pltpu.make_async_copy(my_ga.at[srct_v.at[c]], grows_v.at[p],
                              sem_in[p]).wait()
        pltpu.make_async_copy(my_ad.at[dstt_v.at[c]], adst_v.at[p],
                              sem_in[p]).wait()

    def do_scatter(c, p):
        pltpu.async_copy(s_v.at[p], acc_sh.at[dstt_v.at[c]], sem_sc[p],
                         add=True)

    def wait_sc(c, p):
        pltpu.make_async_copy(s_v.at[p], acc_sh.at[dstt_v.at[c]],
                              sem_sc[p]).wait()

    dnums = lax.GatherDimensionNumbers(
        offset_dims=(), collapsed_slice_dims=(0,), start_index_map=(0,))
    bcast_idx = [jnp.full((16, 1), hh, jnp.int32) for hh in range(HH)]

    def compute(p):
        gp = grows_v.at[p]
        ap = adst_v.at[p]
        lp = alev_v.at[p]
        sp = s_v.at[p]

        @plsc.parallel_loop(0, K, unroll=4)
        def edge(e):
            alev = lp[e, :]
            av = gp[e, pl.ds(64, 16)] + ap[e, :] + alev
            al = jnp.where(av > 0, av, 0.2 * av)
            ex = jnp.exp(al)
            mix = jnp.where(lo4, ex, jnp.where(lo8, alev, one8v))
            sp[e, pl.ds(64, 16)] = mix
            for hh in range(HH):
                exb = lax.gather(ex, bcast_idx[hh], dnums, slice_sizes=(1,),
                                 mode=lax.GatherScatterMode.PROMISE_IN_BOUNDS)
                sp[e, pl.ds(hh * 16, 16)] = gp[e, pl.ds(hh * 16, 16)] * exb

    issue_in(0, 0)
    issue_in(1, 1)
    for c, p in ((0, 0), (1, 1)):
        wait_in(c, p)
        compute(p)
        do_scatter(c, p)
        issue_in(c + 2, p)

    def pair(i, _):
        for off, p in ((0, 0), (1, 1)):
            c = 2 * i + off
            wait_in(c, p)
            wait_sc(c - 2, p)
            compute(p)
            do_scatter(c, p)
            issue_in(c + 2, p)
        return 0
    lax.fori_loop(1, NCHUNK // 2 - 1, pair, 0)

    for c, p in ((NCHUNK - 2, 0), (NCHUNK - 1, 1)):
        wait_in(c, p)
        wait_sc(c - 2, p)
        compute(p)
        do_scatter(c, p)
    wait_sc(NCHUNK - 2, 0)
    wait_sc(NCHUNK - 1, 1)

    plsc.subcore_barrier()
    pltpu.sync_copy(acc_sh.at[pl.ds(sid * 640, 640)],
                    part_h.at[cid, pl.ds(sid * 640, 640)])


def _sc_edge_pass(src3, dst3, ale, ga, ad):
    mesh = plsc.VectorSubcoreMesh(core_axis_name="c", subcore_axis_name="s")
    f = pl.kernel(
        _sc_body,
        mesh=mesh,
        compiler_params=pltpu.CompilerParams(use_tc_tiling_on_sc=False),
        out_type=jax.ShapeDtypeStruct((2, NPAD, ROWS), jnp.float32),
        scratch_types=[
            pltpu.VMEM((NCHUNK, K), jnp.int32),
            pltpu.VMEM((NCHUNK, K), jnp.int32),
            pltpu.VMEM((2, K, ROWG), jnp.float32),
            pltpu.VMEM((2, K, 16), jnp.float32),
            pltpu.VMEM((2, K, 16), jnp.float32),
            pltpu.VMEM((2, K, ROWS), jnp.float32),
            pltpu.VMEM_SHARED((NPAD, ROWS), jnp.float32),
            pltpu.SemaphoreType.DMA,
            pltpu.SemaphoreType.DMA,
            pltpu.SemaphoreType.DMA,
            pltpu.SemaphoreType.DMA,
        ],
    )
    return f(src3, dst3, ale, ga, ad)


def _dup4(a):
    return jnp.concatenate([a, a, a, a], axis=1)


def _prep_w(W, a_src, a_dst):
    w3 = W.reshape(D, H, C)
    ws = jnp.einsum('dhc,hc->dh', w3, a_src)
    wd = jnp.einsum('dhc,hc->dh', w3, a_dst)
    planes = []
    for c in range(2):
        planes.append(jnp.concatenate(
            [W[:, 64 * c:64 * c + 64],
             _dup4(ws[:, HH * c:HH * c + HH]),
             _dup4(wd[:, HH * c:HH * c + HH])], axis=1))
    return jnp.stack(planes)


def _prep_ve(We, a_e):
    ve = jnp.einsum('dhc,hc->dh', We.reshape(DE, H, C), a_e)
    return jnp.stack([_dup4(ve[:, :HH]), _dup4(ve[:, HH:])])


def kernel(nf, ei, ew, W1, as1, ad1, We1, ae1, b1, W2, as2, ad2, We2, ae2, b2,
           g1, bn1, g2, bn2, g3, bn3, Wf1, bf1, Wf2, bf2):
    src3 = ei[0].reshape(16, NCHUNK, K)
    dst3 = ei[1].reshape(16, NCHUNK, K)
    wcat1 = _prep_w(W1, as1, ad1)
    wcat2 = _prep_w(W2, as2, ad2)
    ale1, ale2 = _ale_both(ew, _prep_ve(We1, ae1), _prep_ve(We2, ae2))
    p8 = jnp.repeat(jnp.eye(H, dtype=jnp.float32), C, axis=1)

    ga1 = _proj(nf, wcat1)
    part1 = _sc_edge_pass(src3, dst3, ale1, ga1, ga1[:, :, 80:])
    nf = _post(part1, ga1, nf,
               b1.reshape(1, D), g1.reshape(1, D), bn1.reshape(1, D), p8)

    ga2 = _proj(nf, wcat2)
    part2 = _sc_edge_pass(src3, dst3, ale2, ga2, ga2[:, :, 80:])
    nf = _post_ffn(part2, ga2, nf,
                   b2.reshape(1, D), g2.reshape(1, D), bn2.reshape(1, D), p8,
                   Wf1, bf1.reshape(1, DFF), Wf2, bf2.reshape(1, D),
                   g3.reshape(1, D), bn3.reshape(1, D))
    return nf


# speedup vs baseline: 1.7441x; 1.0215x over previous
import functools
import jax
import jax.numpy as jnp
from jax import lax
from jax.experimental import pallas as pl
from jax.experimental.pallas import tpu as pltpu
from jax.experimental.pallas import tpu_sc as plsc

N = 10000
E = 320000
D = 128
H = 8
C = 16
DE = 16
DFF = 512

NPAD = 10240
HH = H // 2
ROWG = 96
ROWS = 80
K = 80
TILE_E = E // 16
NCHUNK = TILE_E // K
NB = 2000
EB = 8000


def _mm_body(x_ref, w_ref, o_ref):
    o_ref[...] = jnp.dot(x_ref[...], w_ref[...],
                         preferred_element_type=jnp.float32)


def _mm3_body(x_ref, w_ref, o_ref):
    o_ref[0] = jnp.dot(x_ref[...], w_ref[0],
                       preferred_element_type=jnp.float32)


def _proj(nf, wcat):
    return pl.pallas_call(
        _mm3_body,
        grid=(2, N // NB),
        in_specs=[pl.BlockSpec((NB, D), lambda j, i: (i, 0)),
                  pl.BlockSpec((1, D, ROWG), lambda j, i: (j, 0, 0))],
        out_specs=pl.BlockSpec((1, NB, ROWG), lambda j, i: (j, i, 0)),
        out_shape=jax.ShapeDtypeStruct((2, N, ROWG), jnp.float32),
    )(nf, wcat)


def _ale_body(ew_ref, v1_ref, v2_ref, o1_ref, o2_ref):
    ewb = ew_ref[...]
    o1_ref[0] = jnp.dot(ewb, v1_ref[0], preferred_element_type=jnp.float32)
    o2_ref[0] = jnp.dot(ewb, v2_ref[0], preferred_element_type=jnp.float32)


def _ale_both(ew, ve1, ve2):
    return pl.pallas_call(
        _ale_body,
        grid=(2, E // EB),
        in_specs=[pl.BlockSpec((EB, DE), lambda j, i: (i, 0)),
                  pl.BlockSpec((1, DE, 16), lambda j, i: (j, 0, 0)),
                  pl.BlockSpec((1, DE, 16), lambda j, i: (j, 0, 0))],
        out_specs=[pl.BlockSpec((1, EB, 16), lambda j, i: (j, i, 0)),
                   pl.BlockSpec((1, EB, 16), lambda j, i: (j, i, 0))],
        out_shape=[jax.ShapeDtypeStruct((2, E, 16), jnp.float32),
                   jax.ShapeDtypeStruct((2, E, 16), jnp.float32)],
    )(ew, ve1, ve2)


def _ln_leaky(x, gg, bn):
    m = jnp.mean(x, axis=1, keepdims=True)
    xc = x - m
    s = jnp.sqrt(jnp.sum(xc * xc, axis=1, keepdims=True) / (D - 1))
    y = gg * xc / (s + 1e-6) + bn
    return jnp.where(y > 0, y, 0.01 * y)


def _post_core(p0_ref, p1_ref, g0_ref, g1_ref, nf_ref,
               b_ref, gg_ref, bn_ref, p8_ref):
    p0 = p0_ref[0]
    p1 = p1_ref[0]
    g0 = g0_ref[0]
    g1 = g1_ref[0]
    h = jnp.concatenate([g0[:, :64], g1[:, :64]], axis=1)
    als = jnp.concatenate([g0[:, 64:64 + HH], g1[:, 64:64 + HH]], axis=1)
    ald = jnp.concatenate([g0[:, 80:80 + HH], g1[:, 80:80 + HH]], axis=1)
    acc = jnp.concatenate([p0[:, :64], p1[:, :64]], axis=1)
    den_p = jnp.concatenate([p0[:, 64:64 + HH], p1[:, 64:64 + HH]], axis=1)
    acc_la = jnp.concatenate([p0[:, 68:68 + HH], p1[:, 68:68 + HH]], axis=1)
    deg = p0[:, 72:73]
    ale_loop = acc_la / jnp.maximum(deg, 1.0)
    al = als + ald + ale_loop
    al = jnp.where(al > 0, al, 0.2 * al)
    exl = jnp.exp(al)
    rden = 1.0 / (den_p + exl + 1e-16)
    p8 = p8_ref[...]
    exl128 = jnp.dot(exl, p8, preferred_element_type=jnp.float32)
    rden128 = jnp.dot(rden, p8, preferred_element_type=jnp.float32)
    a1 = (acc + exl128 * h) * rden128 + b_ref[...]
    return nf_ref[...] + _ln_leaky(a1, gg_ref[...], bn_ref[...])


def _post_body(p0_ref, p1_ref, g0_ref, g1_ref, nf_ref,
               b_ref, gg_ref, bn_ref, p8_ref, o_ref):
    o_ref[...] = _post_core(p0_ref, p1_ref, g0_ref, g1_ref,
                            nf_ref, b_ref, gg_ref, bn_ref, p8_ref)


def _post_ffn_body(p0_ref, p1_ref, g0_ref, g1_ref, nf_ref,
                   b_ref, gg_ref, bn_ref, p8_ref,
                   w1_ref, b1_ref, w2_ref, b2_ref, gg3_ref, bn3_ref, o_ref):
    nf2 = _post_core(p0_ref, p1_ref, g0_ref, g1_ref,
                     nf_ref, b_ref, gg_ref, bn_ref, p8_ref)
    t = jnp.dot(nf2, w1_ref[...], preferred_element_type=jnp.float32)
    t = jnp.maximum(t + b1_ref[...], 0.0)
    ff = jnp.dot(t, w2_ref[...], preferred_element_type=jnp.float32)
    ff = ff + b2_ref[...]
    o_ref[...] = nf2 + _ln_leaky(ff, gg3_ref[...], bn3_ref[...])


def _post_specs():
    return [pl.BlockSpec((1, NB, ROWS), lambda i: (0, i, 0)),
            pl.BlockSpec((1, NB, ROWS), lambda i: (1, i, 0)),
            pl.BlockSpec((1, NB, ROWG), lambda i: (0, i, 0)),
            pl.BlockSpec((1, NB, ROWG), lambda i: (1, i, 0)),
            pl.BlockSpec((NB, D), lambda i: (i, 0)),
            pl.BlockSpec((1, D), lambda i: (0, 0)),
            pl.BlockSpec((1, D), lambda i: (0, 0)),
            pl.BlockSpec((1, D), lambda i: (0, 0)),
            pl.BlockSpec((H, D), lambda i: (0, 0))]


def _post(part, ga, nf, b, gg, bn, p8):
    return pl.pallas_call(
        _post_body,
        grid=(N // NB,),
        in_specs=_post_specs(),
        out_specs=pl.BlockSpec((NB, D), lambda i: (i, 0)),
        out_shape=jax.ShapeDtypeStruct((N, D), jnp.float32),
    )(part, part, ga, ga, nf, b, gg, bn, p8)


def _post_ffn(part, ga, nf, b, gg, bn, p8, w1, b1, w2, b2, gg3, bn3):
    specs = _post_specs() + [
        pl.BlockSpec((D, DFF), lambda i: (0, 0)),
        pl.BlockSpec((1, DFF), lambda i: (0, 0)),
        pl.BlockSpec((DFF, D), lambda i: (0, 0)),
        pl.BlockSpec((1, D), lambda i: (0, 0)),
        pl.BlockSpec((1, D), lambda i: (0, 0)),
        pl.BlockSpec((1, D), lambda i: (0, 0))]
    return pl.pallas_call(
        _post_ffn_body,
        grid=(N // NB,),
        in_specs=specs,
        out_specs=pl.BlockSpec((NB, D), lambda i: (i, 0)),
        out_shape=jax.ShapeDtypeStruct((N, D), jnp.float32),
    )(part, part, ga, ga, nf, b, gg, bn, p8, w1, b1, w2, b2, gg3, bn3)


def _sc_body(src3_h, dst3_h, ale_h, ga_h, ad_h, part_h,
             srct_v, dstt_v, grows_v, adst_v, alev_v, s_v, acc_sh,
             sem_in0, sem_in1, sem_sc0, sem_sc1):
    cid = lax.axis_index("c")
    sid = lax.axis_index("s")
    lanes = lax.iota(jnp.int32, 16)
    zv = jnp.zeros((16,), jnp.float32)
    one8v = jnp.where(lanes == 8, 1.0, 0.0).astype(jnp.float32)
    lo4 = lanes < 4
    lo8 = lanes < 8
    sem_in = (sem_in0, sem_in1)
    sem_sc = (sem_sc0, sem_sc1)
    estart = sid * TILE_E
    my_ga = ga_h.at[cid]
    my_ad = ad_h.at[cid]
    my_ale = ale_h.at[cid]

    pltpu.sync_copy(src3_h.at[sid], srct_v)
    pltpu.sync_copy(dst3_h.at[sid], dstt_v)

    z_v = s_v.at[0]

    def zrow(i, _):
        r = i // (ROWS // 16)
        col = (i % (ROWS // 16)) * 16
        z_v[r, pl.ds(col, 16)] = zv
        return 0
    lax.fori_loop(0, K * (ROWS // 16), zrow, 0)

    def zcopy(j, _):
        pltpu.sync_copy(z_v, acc_sh.at[pl.ds(sid * 640 + j * K, K)])
        return 0
    lax.fori_loop(0, 640 // K, zcopy, 0)
    plsc.subcore_barrier()

    def issue_in(c, p):
        base = estart + c * K
        pltpu.async_copy(my_ale.at[pl.ds(base, K)], alev_v.at[p], sem_in[p])
        pltpu.async_copy(my_ga.at[srct_v.at[c]], grows_v.at[p], sem_in[p])
        pltpu.async_copy(my_ad.at[dstt_v.at[c]], adst_v.at[p], sem_in[p])

    def wait_in(c, p):
        base = estart + c * K
        pltpu.make_async_copy(my_ale.at[pl.ds(base, K)], alev_v.at[p],
                              sem_in[p]).wait()
        pltpu.make_async_copy(my_ga.at[srct_v.at[c]], grows_v.at[p],
                              sem_in[p]).wait()
        pltpu.make_async_copy(my_ad.at[dstt_v.at[c]], adst_v.at[p],
                              sem_in[p]).wait()

    def do_scatter(c, p):
        pltpu.async_copy(s_v.at[p], acc_sh.at[dstt_v.at[c]], sem_sc[p],
                         add=True)

    def wait_sc(c, p):
        pltpu.make_async_copy(s_v.at[p], acc_sh.at[dstt_v.at[c]],
                              sem_sc[p]).wait()

    dnums = lax.GatherDimensionNumbers(
        offset_dims=(), collapsed_slice_dims=(0,), start_index_map=(0,))
    bcast_idx = [jnp.full((16, 1), hh, jnp.int32) for hh in range(HH)]

    def compute(p):
        gp = grows_v.at[p]
        ap = adst_v.at[p]
        lp = alev_v.at[p]
        sp = s_v.at[p]

        @plsc.parallel_loop(0, K, unroll=8)
        def edge(e):
            alev = lp[e, :]
            av = gp[e, pl.ds(64, 16)] + ap[e, :] + alev
            al = jnp.where(av > 0, av, 0.2 * av)
            ex = jnp.exp(al)
            mix = jnp.where(lo4, ex, jnp.where(lo8, alev, one8v))
            sp[e, pl.ds(64, 16)] = mix
            for hh in range(HH):
                exb = lax.gather(ex, bcast_idx[hh], dnums, slice_sizes=(1,),
                                 mode=lax.GatherScatterMode.PROMISE_IN_BOUNDS)
                sp[e, pl.ds(hh * 16, 16)] = gp[e, pl.ds(hh * 16, 16)] * exb

    issue_in(0, 0)
    issue_in(1, 1)
    for c, p in ((0, 0), (1, 1)):
        wait_in(c, p)
        compute(p)
        do_scatter(c, p)
        issue_in(c + 2, p)

    def pair(i, _):
        for off, p in ((0, 0), (1, 1)):
            c = 2 * i + off
            wait_in(c, p)
            wait_sc(c - 2, p)
            compute(p)
            do_scatter(c, p)
            issue_in(c + 2, p)
        return 0
    lax.fori_loop(1, NCHUNK // 2 - 1, pair, 0)

    for c, p in ((NCHUNK - 2, 0), (NCHUNK - 1, 1)):
        wait_in(c, p)
        wait_sc(c - 2, p)
        compute(p)
        do_scatter(c, p)
    wait_sc(NCHUNK - 2, 0)
    wait_sc(NCHUNK - 1, 1)

    plsc.subcore_barrier()
    pltpu.sync_copy(acc_sh.at[pl.ds(sid * 640, 640)],
                    part_h.at[cid, pl.ds(sid * 640, 640)])


def _sc_edge_pass(src3, dst3, ale, ga, ad):
    mesh = plsc.VectorSubcoreMesh(core_axis_name="c", subcore_axis_name="s")
    f = pl.kernel(
        _sc_body,
        mesh=mesh,
        compiler_params=pltpu.CompilerParams(use_tc_tiling_on_sc=False),
        out_type=jax.ShapeDtypeStruct((2, NPAD, ROWS), jnp.float32),
        scratch_types=[
            pltpu.VMEM((NCHUNK, K), jnp.int32),
            pltpu.VMEM((NCHUNK, K), jnp.int32),
            pltpu.VMEM((2, K, ROWG), jnp.float32),
            pltpu.VMEM((2, K, 16), jnp.float32),
            pltpu.VMEM((2, K, 16), jnp.float32),
            pltpu.VMEM((2, K, ROWS), jnp.float32),
            pltpu.VMEM_SHARED((NPAD, ROWS), jnp.float32),
            pltpu.SemaphoreType.DMA,
            pltpu.SemaphoreType.DMA,
            pltpu.SemaphoreType.DMA,
            pltpu.SemaphoreType.DMA,
        ],
    )
    return f(src3, dst3, ale, ga, ad)


def _dup4(a):
    return jnp.concatenate([a, a, a, a], axis=1)


def _prep_w(W, a_src, a_dst):
    w3 = W.reshape(D, H, C)
    ws = jnp.einsum('dhc,hc->dh', w3, a_src)
    wd = jnp.einsum('dhc,hc->dh', w3, a_dst)
    planes = []
    for c in range(2):
        planes.append(jnp.concatenate(
            [W[:, 64 * c:64 * c + 64],
             _dup4(ws[:, HH * c:HH * c + HH]),
             _dup4(wd[:, HH * c:HH * c + HH])], axis=1))
    return jnp.stack(planes)


def _prep_ve(We, a_e):
    ve = jnp.einsum('dhc,hc->dh', We.reshape(DE, H, C), a_e)
    return jnp.stack([_dup4(ve[:, :HH]), _dup4(ve[:, HH:])])


def kernel(nf, ei, ew, W1, as1, ad1, We1, ae1, b1, W2, as2, ad2, We2, ae2, b2,
           g1, bn1, g2, bn2, g3, bn3, Wf1, bf1, Wf2, bf2):
    src3 = ei[0].reshape(16, NCHUNK, K)
    dst3 = ei[1].reshape(16, NCHUNK, K)
    wcat1 = _prep_w(W1, as1, ad1)
    wcat2 = _prep_w(W2, as2, ad2)
    ale1, ale2 = _ale_both(ew, _prep_ve(We1, ae1), _prep_ve(We2, ae2))
    p8 = jnp.repeat(jnp.eye(H, dtype=jnp.float32), C, axis=1)

    ga1 = _proj(nf, wcat1)
    part1 = _sc_edge_pass(src3, dst3, ale1, ga1, ga1[:, :, 80:])
    nf = _post(part1, ga1, nf,
               b1.reshape(1, D), g1.reshape(1, D), bn1.reshape(1, D), p8)

    ga2 = _proj(nf, wcat2)
    part2 = _sc_edge_pass(src3, dst3, ale2, ga2, ga2[:, :, 80:])
    nf = _post_ffn(part2, ga2, nf,
                   b2.reshape(1, D), g2.reshape(1, D), bn2.reshape(1, D), p8,
                   Wf1, bf1.reshape(1, DFF), Wf2, bf2.reshape(1, D),
                   g3.reshape(1, D), bn3.reshape(1, D))
    return nf


# speedup vs baseline: 1.7446x; 1.0003x over previous
import functools
import jax
import jax.numpy as jnp
from jax import lax
from jax.experimental import pallas as pl
from jax.experimental.pallas import tpu as pltpu
from jax.experimental.pallas import tpu_sc as plsc

N = 10000
E = 320000
D = 128
H = 8
C = 16
DE = 16
DFF = 512

NPAD = 10240
HH = H // 2
ROWG = 96
ROWS = 80
K = 80
TILE_E = E // 16
NCHUNK = TILE_E // K
NB = 2000
EB = 8000


def _mm_body(x_ref, w_ref, o_ref):
    o_ref[...] = jnp.dot(x_ref[...], w_ref[...],
                         preferred_element_type=jnp.float32)


def _mm3_body(x_ref, w_ref, o_ref):
    o_ref[0] = jnp.dot(x_ref[...], w_ref[0],
                       preferred_element_type=jnp.float32)


def _proj(nf, wcat):
    return pl.pallas_call(
        _mm3_body,
        grid=(2, N // NB),
        in_specs=[pl.BlockSpec((NB, D), lambda j, i: (i, 0)),
                  pl.BlockSpec((1, D, ROWG), lambda j, i: (j, 0, 0))],
        out_specs=pl.BlockSpec((1, NB, ROWG), lambda j, i: (j, i, 0)),
        out_shape=jax.ShapeDtypeStruct((2, N, ROWG), jnp.float32),
    )(nf, wcat)


def _ale_body(ew_ref, v1_ref, v2_ref, o1_ref, o2_ref):
    ewb = ew_ref[...]
    o1_ref[0] = jnp.dot(ewb, v1_ref[0], preferred_element_type=jnp.float32)
    o2_ref[0] = jnp.dot(ewb, v2_ref[0], preferred_element_type=jnp.float32)


def _ale_both(ew, ve1, ve2):
    return pl.pallas_call(
        _ale_body,
        grid=(2, E // EB),
        in_specs=[pl.BlockSpec((EB, DE), lambda j, i: (i, 0)),
                  pl.BlockSpec((1, DE, 16), lambda j, i: (j, 0, 0)),
                  pl.BlockSpec((1, DE, 16), lambda j, i: (j, 0, 0))],
        out_specs=[pl.BlockSpec((1, EB, 16), lambda j, i: (j, i, 0)),
                   pl.BlockSpec((1, EB, 16), lambda j, i: (j, i, 0))],
        out_shape=[jax.ShapeDtypeStruct((2, E, 16), jnp.float32),
                   jax.ShapeDtypeStruct((2, E, 16), jnp.float32)],
    )(ew, ve1, ve2)


def _ln_leaky(x, gg, bn):
    m = jnp.mean(x, axis=1, keepdims=True)
    xc = x - m
    s = jnp.sqrt(jnp.sum(xc * xc, axis=1, keepdims=True) / (D - 1))
    y = gg * xc / (s + 1e-6) + bn
    return jnp.where(y > 0, y, 0.01 * y)


def _post_core(p0_ref, p1_ref, g0_ref, g1_ref, nf_ref,
               b_ref, gg_ref, bn_ref, p8_ref):
    p0 = p0_ref[0]
    p1 = p1_ref[0]
    g0 = g0_ref[0]
    g1 = g1_ref[0]
    h = jnp.concatenate([g0[:, :64], g1[:, :64]], axis=1)
    als = jnp.concatenate([g0[:, 64:64 + HH], g1[:, 64:64 + HH]], axis=1)
    ald = jnp.concatenate([g0[:, 80:80 + HH], g1[:, 80:80 + HH]], axis=1)
    acc = jnp.concatenate([p0[:, :64], p1[:, :64]], axis=1)
    den_p = jnp.concatenate([p0[:, 64:64 + HH], p1[:, 64:64 + HH]], axis=1)
    acc_la = jnp.concatenate([p0[:, 68:68 + HH], p1[:, 68:68 + HH]], axis=1)
    deg = p0[:, 72:73]
    ale_loop = acc_la / jnp.maximum(deg, 1.0)
    al = als + ald + ale_loop
    al = jnp.where(al > 0, al, 0.2 * al)
    exl = jnp.exp(al)
    rden = 1.0 / (den_p + exl + 1e-16)
    p8 = p8_ref[...]
    exl128 = jnp.dot(exl, p8, preferred_element_type=jnp.float32)
    rden128 = jnp.dot(rden, p8, preferred_element_type=jnp.float32)
    a1 = (acc + exl128 * h) * rden128 + b_ref[...]
    return nf_ref[...] + _ln_leaky(a1, gg_ref[...], bn_ref[...])


def _post_body(p0_ref, p1_ref, g0_ref, g1_ref, nf_ref,
               b_ref, gg_ref, bn_ref, p8_ref, o_ref):
    o_ref[...] = _post_core(p0_ref, p1_ref, g0_ref, g1_ref,
                            nf_ref, b_ref, gg_ref, bn_ref, p8_ref)


def _post_ffn_body(p0_ref, p1_ref, g0_ref, g1_ref, nf_ref,
                   b_ref, gg_ref, bn_ref, p8_ref,
                   w1_ref, b1_ref, w2_ref, b2_ref, gg3_ref, bn3_ref, o_ref):
    nf2 = _post_core(p0_ref, p1_ref, g0_ref, g1_ref,
                     nf_ref, b_ref, gg_ref, bn_ref, p8_ref)
    t = jnp.dot(nf2, w1_ref[...], preferred_element_type=jnp.float32)
    t = jnp.maximum(t + b1_ref[...], 0.0)
    ff = jnp.dot(t, w2_ref[...], preferred_element_type=jnp.float32)
    ff = ff + b2_ref[...]
    o_ref[...] = nf2 + _ln_leaky(ff, gg3_ref[...], bn3_ref[...])


def _post_specs():
    return [pl.BlockSpec((1, NB, ROWS), lambda i: (0, i, 0)),
            pl.BlockSpec((1, NB, ROWS), lambda i: (1, i, 0)),
            pl.BlockSpec((1, NB, ROWG), lambda i: (0, i, 0)),
            pl.BlockSpec((1, NB, ROWG), lambda i: (1, i, 0)),
            pl.BlockSpec((NB, D), lambda i: (i, 0)),
            pl.BlockSpec((1, D), lambda i: (0, 0)),
            pl.BlockSpec((1, D), lambda i: (0, 0)),
            pl.BlockSpec((1, D), lambda i: (0, 0)),
            pl.BlockSpec((H, D), lambda i: (0, 0))]


def _post(part, ga, nf, b, gg, bn, p8):
    return pl.pallas_call(
        _post_body,
        grid=(N // NB,),
        in_specs=_post_specs(),
        out_specs=pl.BlockSpec((NB, D), lambda i: (i, 0)),
        out_shape=jax.ShapeDtypeStruct((N, D), jnp.float32),
    )(part, part, ga, ga, nf, b, gg, bn, p8)


def _post_ffn(part, ga, nf, b, gg, bn, p8, w1, b1, w2, b2, gg3, bn3):
    specs = _post_specs() + [
        pl.BlockSpec((D, DFF), lambda i: (0, 0)),
        pl.BlockSpec((1, DFF), lambda i: (0, 0)),
        pl.BlockSpec((DFF, D), lambda i: (0, 0)),
        pl.BlockSpec((1, D), lambda i: (0, 0)),
        pl.BlockSpec((1, D), lambda i: (0, 0)),
        pl.BlockSpec((1, D), lambda i: (0, 0))]
    return pl.pallas_call(
        _post_ffn_body,
        grid=(N // NB,),
        in_specs=specs,
        out_specs=pl.BlockSpec((NB, D), lambda i: (i, 0)),
        out_shape=jax.ShapeDtypeStruct((N, D), jnp.float32),
    )(part, part, ga, ga, nf, b, gg, bn, p8, w1, b1, w2, b2, gg3, bn3)


def _sc_body(src3_h, dst3_h, ale_h, ga_h, ad_h, part_h,
             srct_v, dstt_v, grows_v, adst_v, alev_v, s_v, acc_sh,
             sem_in0, sem_in1, sem_sc0, sem_sc1):
    cid = lax.axis_index("c")
    sid = lax.axis_index("s")
    lanes = lax.iota(jnp.int32, 16)
    zv = jnp.zeros((16,), jnp.float32)
    one8v = jnp.where(lanes == 8, 1.0, 0.0).astype(jnp.float32)
    lo4 = lanes < 4
    lo8 = lanes < 8
    sem_in = (sem_in0, sem_in1)
    sem_sc = (sem_sc0, sem_sc1)
    estart = sid * TILE_E
    my_ga = ga_h.at[cid]
    my_ad = ad_h.at[cid]
    my_ale = ale_h.at[cid]

    pltpu.sync_copy(src3_h.at[sid], srct_v)
    pltpu.sync_copy(dst3_h.at[sid], dstt_v)

    z_v = s_v.at[0]

    @plsc.parallel_loop(0, K * (ROWS // 16), unroll=4)
    def zrow(i):
        r = i // (ROWS // 16)
        col = (i % (ROWS // 16)) * 16
        z_v[r, pl.ds(col, 16)] = zv

    def zcopy(j, _):
        pltpu.sync_copy(z_v, acc_sh.at[pl.ds(sid * 640 + j * K, K)])
        return 0
    lax.fori_loop(0, 640 // K, zcopy, 0)
    plsc.subcore_barrier()

    def issue_in(c, p):
        base = estart + c * K
        pltpu.async_copy(my_ale.at[pl.ds(base, K)], alev_v.at[p], sem_in[p])
        pltpu.async_copy(my_ga.at[srct_v.at[c]], grows_v.at[p], sem_in[p])
        pltpu.async_copy(my_ad.at[dstt_v.at[c]], adst_v.at[p], sem_in[p])

    def wait_in(c, p):
        base = estart + c * K
        pltpu.make_async_copy(my_ale.at[pl.ds(base, K)], alev_v.at[p],
                              sem_in[p]).wait()
        pltpu.make_async_copy(my_ga.at[srct_v.at[c]], grows_v.at[p],
                              sem_in[p]).wait()
        pltpu.make_async_copy(my_ad.at[dstt_v.at[c]], adst_v.at[p],
                              sem_in[p]).wait()

    def do_scatter(c, p):
        pltpu.async_copy(s_v.at[p], acc_sh.at[dstt_v.at[c]], sem_sc[p],
                         add=True)

    def wait_sc(c, p):
        pltpu.make_async_copy(s_v.at[p], acc_sh.at[dstt_v.at[c]],
                              sem_sc[p]).wait()

    dnums = lax.GatherDimensionNumbers(
        offset_dims=(), collapsed_slice_dims=(0,), start_index_map=(0,))
    bcast_idx = [jnp.full((16, 1), hh, jnp.int32) for hh in range(HH)]

    def compute(p):
        gp = grows_v.at[p]
        ap = adst_v.at[p]
        lp = alev_v.at[p]
        sp = s_v.at[p]

        @plsc.parallel_loop(0, K, unroll=8)
        def edge(e):
            alev = lp[e, :]
            av = gp[e, pl.ds(64, 16)] + ap[e, :] + alev
            al = jnp.where(av > 0, av, 0.2 * av)
            ex = jnp.exp(al)
            mix = jnp.where(lo4, ex, jnp.where(lo8, alev, one8v))
            sp[e, pl.ds(64, 16)] = mix
            for hh in range(HH):
                exb = lax.gather(ex, bcast_idx[hh], dnums, slice_sizes=(1,),
                                 mode=lax.GatherScatterMode.PROMISE_IN_BOUNDS)
                sp[e, pl.ds(hh * 16, 16)] = gp[e, pl.ds(hh * 16, 16)] * exb

    issue_in(0, 0)
    issue_in(1, 1)
    for c, p in ((0, 0), (1, 1)):
        wait_in(c, p)
        compute(p)
        do_scatter(c, p)
        issue_in(c + 2, p)

    def pair(i, _):
        for off, p in ((0, 0), (1, 1)):
            c = 2 * i + off
            wait_in(c, p)
            wait_sc(c - 2, p)
            compute(p)
            do_scatter(c, p)
            issue_in(c + 2, p)
        return 0
    lax.fori_loop(1, NCHUNK // 2 - 1, pair, 0)

    for c, p in ((NCHUNK - 2, 0), (NCHUNK - 1, 1)):
        wait_in(c, p)
        wait_sc(c - 2, p)
        compute(p)
        do_scatter(c, p)
    wait_sc(NCHUNK - 2, 0)
    wait_sc(NCHUNK - 1, 1)

    plsc.subcore_barrier()
    pltpu.sync_copy(acc_sh.at[pl.ds(sid * 640, 640)],
                    part_h.at[cid, pl.ds(sid * 640, 640)])


def _sc_edge_pass(src3, dst3, ale, ga, ad):
    mesh = plsc.VectorSubcoreMesh(core_axis_name="c", subcore_axis_name="s")
    f = pl.kernel(
        _sc_body,
        mesh=mesh,
        compiler_params=pltpu.CompilerParams(use_tc_tiling_on_sc=False),
        out_type=jax.ShapeDtypeStruct((2, NPAD, ROWS), jnp.float32),
        scratch_types=[
            pltpu.VMEM((NCHUNK, K), jnp.int32),
            pltpu.VMEM((NCHUNK, K), jnp.int32),
            pltpu.VMEM((2, K, ROWG), jnp.float32),
            pltpu.VMEM((2, K, 16), jnp.float32),
            pltpu.VMEM((2, K, 16), jnp.float32),
            pltpu.VMEM((2, K, ROWS), jnp.float32),
            pltpu.VMEM_SHARED((NPAD, ROWS), jnp.float32),
            pltpu.SemaphoreType.DMA,
            pltpu.SemaphoreType.DMA,
            pltpu.SemaphoreType.DMA,
            pltpu.SemaphoreType.DMA,
        ],
    )
    return f(src3, dst3, ale, ga, ad)


def _dup4(a):
    return jnp.concatenate([a, a, a, a], axis=1)


def _prep_w(W, a_src, a_dst):
    w3 = W.reshape(D, H, C)
    ws = jnp.einsum('dhc,hc->dh', w3, a_src)
    wd = jnp.einsum('dhc,hc->dh', w3, a_dst)
    planes = []
    for c in range(2):
        planes.append(jnp.concatenate(
            [W[:, 64 * c:64 * c + 64],
             _dup4(ws[:, HH * c:HH * c + HH]),
             _dup4(wd[:, HH * c:HH * c + HH])], axis=1))
    return jnp.stack(planes)


def _prep_ve(We, a_e):
    ve = jnp.einsum('dhc,hc->dh', We.reshape(DE, H, C), a_e)
    return jnp.stack([_dup4(ve[:, :HH]), _dup4(ve[:, HH:])])


def kernel(nf, ei, ew, W1, as1, ad1, We1, ae1, b1, W2, as2, ad2, We2, ae2, b2,
           g1, bn1, g2, bn2, g3, bn3, Wf1, bf1, Wf2, bf2):
    src3 = ei[0].reshape(16, NCHUNK, K)
    dst3 = ei[1].reshape(16, NCHUNK, K)
    wcat1 = _prep_w(W1, as1, ad1)
    wcat2 = _prep_w(W2, as2, ad2)
    ale1, ale2 = _ale_both(ew, _prep_ve(We1, ae1), _prep_ve(We2, ae2))
    p8 = jnp.repeat(jnp.eye(H, dtype=jnp.float32), C, axis=1)

    ga1 = _proj(nf, wcat1)
    part1 = _sc_edge_pass(src3, dst3, ale1, ga1, ga1[:, :, 80:])
    nf = _post(part1, ga1, nf,
               b1.reshape(1, D), g1.reshape(1, D), bn1.reshape(1, D), p8)

    ga2 = _proj(nf, wcat2)
    part2 = _sc_edge_pass(src3, dst3, ale2, ga2, ga2[:, :, 80:])
    nf = _post_ffn(part2, ga2, nf,
                   b2.reshape(1, D), g2.reshape(1, D), bn2.reshape(1, D), p8,
                   Wf1, bf1.reshape(1, DFF), Wf2, bf2.reshape(1, D),
                   g3.reshape(1, D), bn3.reshape(1, D))
    return nf
